# Initial kernel scaffold; baseline (speedup 1.0000x reference)
import jax
import jax.numpy as jnp
from jax.experimental import pallas as pl


def kernel(indices, embeddings):
    raise NotImplementedError("write your pallas kernel here")


# speedup vs baseline: 4.6108x; 4.6108x over previous
import functools

import jax
import jax.numpy as jnp
from jax import lax
from jax.experimental import pallas as pl
from jax.experimental.pallas import tpu as pltpu
from jax.experimental.pallas import tpu_sc as plsc

BATCH = 4096
SEQ_LEN = 50
NUM_RESIDUES = 1000
EMBED_DIM = 64
OOV_INDEX = 0

NUM_WORKERS = 32
CHUNK = 128
TOTAL = BATCH * SEQ_LEN
ROWS_PER_W = TOTAL // (NUM_WORKERS * CHUNK)


def _sc_gather(idx2d, table):
    mesh = plsc.VectorSubcoreMesh(core_axis_name="c", subcore_axis_name="s")

    @functools.partial(
        pl.kernel,
        mesh=mesh,
        compiler_params=pltpu.CompilerParams(use_tc_tiling_on_sc=False),
        out_type=jax.ShapeDtypeStruct((TOTAL, EMBED_DIM), jnp.float32),
        scratch_types=[
            pltpu.VMEM((ROWS_PER_W, CHUNK), jnp.int32),
            pltpu.VMEM((CHUNK, EMBED_DIM), jnp.float32),
            pltpu.SemaphoreType.DMA,
        ],
    )
    def k(idx_hbm, table_hbm, out_hbm, idx_v, rows_v, sem):
        wid = lax.axis_index("s") * 2 + lax.axis_index("c")
        rbase = wid * ROWS_PER_W
        pltpu.sync_copy(idx_hbm.at[wid], idx_v)

        def body(j, carry):
            pltpu.async_copy(table_hbm.at[idx_v.at[j]], rows_v, sem).wait()
            pltpu.sync_copy(rows_v, out_hbm.at[pl.ds((rbase + j) * CHUNK, CHUNK)])
            return carry

        lax.fori_loop(0, ROWS_PER_W, body, 0)

    return k(idx2d, table)


def kernel(indices, embeddings):
    idx = jnp.where(indices == -1, OOV_INDEX, indices)
    idx3d = idx.reshape(NUM_WORKERS, ROWS_PER_W, CHUNK)
    out = _sc_gather(idx3d, embeddings)
    return out.reshape(BATCH, SEQ_LEN, EMBED_DIM)


# speedup vs baseline: 4.8947x; 1.0616x over previous
import functools

import jax
import jax.numpy as jnp
from jax import lax
from jax.experimental import pallas as pl
from jax.experimental.pallas import tpu as pltpu
from jax.experimental.pallas import tpu_sc as plsc

BATCH = 4096
SEQ_LEN = 50
NUM_RESIDUES = 1000
EMBED_DIM = 64
OOV_INDEX = 0

NUM_WORKERS = 32
TOTAL = BATCH * SEQ_LEN
PER_W = TOTAL // NUM_WORKERS
NBUF = 4
GROUPS = 16
GR = PER_W // GROUPS


def _sc_gather(idx2d, table):
    mesh = plsc.VectorSubcoreMesh(core_axis_name="c", subcore_axis_name="s")

    @functools.partial(
        pl.kernel,
        mesh=mesh,
        compiler_params=pltpu.CompilerParams(use_tc_tiling_on_sc=False),
        out_type=jax.ShapeDtypeStruct((TOTAL, EMBED_DIM), jnp.float32),
        scratch_types=[
            pltpu.VMEM((PER_W,), jnp.int32),
            pltpu.VMEM((NBUF, GR, EMBED_DIM), jnp.float32),
        ]
        + [pltpu.SemaphoreType.DMA] * (2 * NBUF),
    )
    def k(idx_hbm, table_hbm, out_hbm, idx_v, rows_v, *sems):
        gsem, osem = sems[:NBUF], sems[NBUF:]
        wid = lax.axis_index("s") * 2 + lax.axis_index("c")
        base = wid * PER_W
        pltpu.sync_copy(idx_hbm.at[wid], idx_v)

        def fire_gather(g, b):
            pltpu.async_copy(
                table_hbm.at[idx_v.at[pl.ds(g * GR, GR)]], rows_v.at[b], gsem[b]
            )

        def wait_gather(b):
            pltpu.make_async_copy(
                table_hbm.at[pl.ds(0, GR)], rows_v.at[b], gsem[b]
            ).wait()

        def fire_scatter(g, b):
            pltpu.async_copy(
                rows_v.at[b], out_hbm.at[pl.ds(base + g * GR, GR)], osem[b]
            )

        def wait_scatter(b):
            pltpu.make_async_copy(
                rows_v.at[b], out_hbm.at[pl.ds(base, GR)], osem[b]
            ).wait()

        for b in range(NBUF):
            fire_gather(b, b)

        def body(i, carry):
            g0 = i * NBUF
            for b in range(NBUF):
                g = g0 + b
                wait_gather(b)
                fire_scatter(g, b)
                wait_scatter(b)
                fire_gather(g + NBUF, b)
            return carry

        lax.fori_loop(0, GROUPS // NBUF - 1, body, 0)

        for b in range(NBUF):
            g = GROUPS - NBUF + b
            wait_gather(b)
            fire_scatter(g, b)
        for b in range(NBUF):
            wait_scatter(b)

    return k(idx2d, table)


def kernel(indices, embeddings):
    idx = jnp.where(indices == -1, OOV_INDEX, indices)
    idx2d = idx.reshape(NUM_WORKERS, PER_W)
    out = _sc_gather(idx2d, embeddings)
    return out.reshape(BATCH, SEQ_LEN, EMBED_DIM)
